# Initial kernel scaffold; baseline (speedup 1.0000x reference)
import jax
import jax.numpy as jnp
from jax.experimental import pallas as pl


def kernel(words_pretrained, table):
    raise NotImplementedError("write your pallas kernel here")


# speedup vs baseline: 1.4780x; 1.4780x over previous
import functools

import jax
import jax.numpy as jnp
from jax import lax
from jax.experimental import pallas as pl
from jax.experimental.pallas import tpu as pltpu
from jax.experimental.pallas import tpu_sc as plsc

NC = 2
NS = 16
NW = NC * NS
D = 32


@functools.lru_cache(maxsize=None)
def _gather_call(B: int):
    b_per_w = B // NW
    chunk = 1024
    nch = b_per_w // chunk
    mesh = plsc.VectorSubcoreMesh(core_axis_name="c", subcore_axis_name="s")

    @functools.partial(
        pl.kernel,
        mesh=mesh,
        out_type=jax.ShapeDtypeStruct((B, D), jnp.float32),
        scratch_types=[
            pltpu.VMEM((b_per_w,), jnp.int32),
            pltpu.VMEM((chunk, D), jnp.float32),
            pltpu.SemaphoreType.DMA,
        ],
        compiler_params=pltpu.CompilerParams(use_tc_tiling_on_sc=False),
    )
    def k(idx_hbm, table_hbm, out_hbm, idx_v, rows_v, sem):
        wid = lax.axis_index("s") * NC + lax.axis_index("c")
        base = wid * b_per_w
        pltpu.sync_copy(idx_hbm.at[pl.ds(base, b_per_w)], idx_v)

        def body(g, carry):
            off = pl.multiple_of(g * chunk, chunk)
            pltpu.async_copy(
                table_hbm.at[idx_v.at[pl.ds(off, chunk)]], rows_v, sem
            ).wait()
            pltpu.sync_copy(rows_v, out_hbm.at[pl.ds(base + off, chunk)])
            return carry

        lax.fori_loop(0, nch, body, 0)

    return k


def kernel(words_pretrained, table):
    b0, hist = words_pretrained.shape
    idx = words_pretrained.reshape(-1).astype(jnp.int32)
    out = _gather_call(idx.shape[0])(idx, table)
    return out.reshape(b0, hist, D)


# speedup vs baseline: 1.4961x; 1.0122x over previous
import functools

import jax
import jax.numpy as jnp
from jax import lax
from jax.experimental import pallas as pl
from jax.experimental.pallas import tpu as pltpu
from jax.experimental.pallas import tpu_sc as plsc

NC = 2
NS = 16
NW = NC * NS
D = 32
CHUNK = 1280


@functools.lru_cache(maxsize=None)
def _gather_call(B: int):
    b_per_w = B // NW
    nch = b_per_w // CHUNK
    assert nch % 2 == 0 and nch >= 4
    mesh = plsc.VectorSubcoreMesh(core_axis_name="c", subcore_axis_name="s")

    @functools.partial(
        pl.kernel,
        mesh=mesh,
        out_type=jax.ShapeDtypeStruct((B, D), jnp.float32),
        scratch_types=[
            pltpu.VMEM((b_per_w,), jnp.int32),
            pltpu.VMEM((CHUNK, D), jnp.float32),
            pltpu.VMEM((CHUNK, D), jnp.float32),
            pltpu.SemaphoreType.DMA,
            pltpu.SemaphoreType.DMA,
        ],
        compiler_params=pltpu.CompilerParams(use_tc_tiling_on_sc=False),
    )
    def k(idx_hbm, table_hbm, out_hbm, idx_v, rows0, rows1, gsem, ssem):
        wid = lax.axis_index("s") * NC + lax.axis_index("c")
        base = wid * b_per_w
        pltpu.sync_copy(idx_hbm.at[pl.ds(base, b_per_w)], idx_v)
        bufs = (rows0, rows1)

        def gather(g, buf):
            off = pl.multiple_of(g * CHUNK, 128)
            return pltpu.make_async_copy(
                table_hbm.at[idx_v.at[pl.ds(off, CHUNK)]], buf, gsem
            )

        def store(g, buf):
            off = pl.multiple_of(base + g * CHUNK, 128)
            return pltpu.make_async_copy(buf, out_hbm.at[pl.ds(off, CHUNK)], ssem)

        gather(0, bufs[0]).start()
        gather(0, bufs[0]).wait()
        gather(1, bufs[1]).start()
        store(0, bufs[0]).start()

        def body(i, carry):
            for p, dg in ((1, 1), (0, 2)):
                g = i * 2 + dg
                gather(g, bufs[p]).wait()
                store(g - 1, bufs[1 - p]).wait()
                gather(g + 1, bufs[1 - p]).start()
                store(g, bufs[p]).start()
            return carry

        lax.fori_loop(0, (nch - 2) // 2, body, 0)

        g = nch - 1
        gather(g, bufs[1]).wait()
        store(g - 1, bufs[0]).wait()
        store(g, bufs[1]).start()
        store(g, bufs[1]).wait()

    return k


def kernel(words_pretrained, table):
    b0, hist = words_pretrained.shape
    idx = words_pretrained.reshape(-1).astype(jnp.int32)
    out = _gather_call(idx.shape[0])(idx, table)
    return out.reshape(b0, hist, D)
